# baseline (speedup 1.0000x reference)
import functools

import jax
import jax.numpy as jnp
from jax import lax
from jax.experimental import pallas as pl
from jax.experimental.pallas import tpu as pltpu
from jax.experimental.pallas import tpu_sc as plsc

WIN = 24
NH = 16
AREA = WIN * WIN
N = AREA + 1
NN = N * N
NREL = (2 * WIN - 1) * (2 * WIN - 1) + 3
B = 16

RCH = 8
NRC = 72
CW = RCH * N
BPG = 8
IDX_PAD = ((NN + 127) // 128) * 128


@functools.cache
def _make_sc_fused():
    mesh = plsc.VectorSubcoreMesh(core_axis_name="c", subcore_axis_name="s")

    @functools.partial(
        pl.kernel,
        mesh=mesh,
        out_type=jax.ShapeDtypeStruct((B, NH, N, N), jnp.float32),
        scratch_types=[
            pltpu.VMEM((NREL * NH,), jnp.float32),
            pltpu.VMEM((BPG, RCH, N), jnp.float32),
            pltpu.VMEM((BPG, RCH, N), jnp.float32),
            pltpu.VMEM((CW,), jnp.int32),
            pltpu.VMEM((CW,), jnp.int32),
            pltpu.SemaphoreType.DMA,
            pltpu.SemaphoreType.DMA,
            pltpu.SemaphoreType.DMA,
            pltpu.SemaphoreType.DMA,
        ],
        compiler_params=pltpu.CompilerParams(needs_layout_passes=False),
    )
    def _sc_fused(attn_hbm, table_hbm, idx_hbm, out_hbm,
                  table_v, ab0, ab1, ib0, ib1,
                  sin0, sin1, sout0, sout1):
        wid = lax.axis_index("s") * 2 + lax.axis_index("c")
        h = wid % NH
        b0 = (wid // NH) * BPG

        pltpu.sync_copy(table_hbm, table_v)

        def attn_src(c):
            return attn_hbm.at[pl.ds(b0, BPG), h, pl.ds(c * RCH, RCH), :]

        def out_dst(c):
            return out_hbm.at[pl.ds(b0, BPG), h, pl.ds(c * RCH, RCH), :]

        def idx_src(c):
            return idx_hbm.at[pl.ds(c * CW, CW)]

        def issue_in(c, ab, ib, sem):
            pltpu.make_async_copy(attn_src(c), ab, sem).start()
            pltpu.make_async_copy(idx_src(c), ib, sem).start()

        def wait_in(c, ab, ib, sem):
            pltpu.make_async_copy(attn_src(c), ab, sem).wait()
            pltpu.make_async_copy(idx_src(c), ib, sem).wait()

        lane = lax.iota(jnp.int32, 16)
        tmask = lane == 15

        def compute(ab, ib):
            def cbody(ci, _):
                co = ci * 16
                for r in range(RCH):
                    off = r * N
                    vidx = ib[pl.ds(off + co, 16)]
                    vb = plsc.load_gather(table_v, [vidx * NH + h])
                    for bi in range(BPG):
                        ab[bi, r, pl.ds(co, 16)] = (
                            ab[bi, r, pl.ds(co, 16)] + vb)
                return _
            lax.fori_loop(0, 36, cbody, None)
            for r in range(RCH):
                off = r * N
                vidx = ib[pl.ds(off + 561, 16)]
                vb = plsc.load_gather(table_v, [vidx * NH + h])
                vbm = jnp.where(tmask, vb, 0.0)
                for bi in range(BPG):
                    ab[bi, r, pl.ds(561, 16)] = (
                        ab[bi, r, pl.ds(561, 16)] + vbm)

        issue_in(0, ab0, ib0, sin0)
        issue_in(1, ab1, ib1, sin1)

        def body(i, _):
            c0 = 2 * i
            c1 = c0 + 1
            wait_in(c0, ab0, ib0, sin0)
            compute(ab0, ib0)
            pltpu.make_async_copy(ab0, out_dst(c0), sout0).start()
            pltpu.make_async_copy(ab0, out_dst(c0), sout0).wait()

            @pl.when(i < NRC // 2 - 1)
            def _():
                issue_in(c0 + 2, ab0, ib0, sin0)

            wait_in(c1, ab1, ib1, sin1)
            compute(ab1, ib1)
            pltpu.make_async_copy(ab1, out_dst(c1), sout1).start()
            pltpu.make_async_copy(ab1, out_dst(c1), sout1).wait()

            @pl.when(i < NRC // 2 - 1)
            def _():
                issue_in(c1 + 2, ab1, ib1, sin1)

            return _

        lax.fori_loop(0, NRC // 2, body, None)

        @pl.when(b0 == 0)
        def _():
            pltpu.sync_copy(idx_hbm.at[pl.ds((N - 1) * N, 584)],
                            ib0.at[pl.ds(0, 584)])
            rbuf = ab0.at[0, pl.ds(0, 1), :]
            for b in range(B):
                pltpu.sync_copy(attn_hbm.at[b, h, pl.ds(N - 1, 1), :], rbuf)
                for ci in range(37):
                    co = min(ci * 16, 561)
                    vidx = ib0[pl.ds(co, 16)]
                    vb = plsc.load_gather(table_v, [vidx * NH + h])
                    if ci == 36:
                        vb = jnp.where(tmask, vb, 0.0)
                    ab0[0, 0, pl.ds(co, 16)] = ab0[0, 0, pl.ds(co, 16)] + vb
                pltpu.sync_copy(rbuf, out_hbm.at[b, h, pl.ds(N - 1, 1), :])

    return _sc_fused


def kernel(attn, relative_position_bias_table, relative_position_index):
    idx = relative_position_index.reshape(-1).astype(jnp.int32)
    idx_pad = jnp.zeros((IDX_PAD,), jnp.int32).at[:NN].set(idx)
    return _make_sc_fused()(attn,
                            relative_position_bias_table.reshape(-1),
                            idx_pad)
